# baseline (speedup 1.0000x reference)
import functools

import jax
import jax.numpy as jnp
from jax import lax
from jax.experimental import pallas as pl
from jax.experimental.pallas import tpu as pltpu
from jax.experimental.pallas import tpu_sc as plsc

VOCAB = 1000
NGRAM = 4
DIM = 32
BATCH = 1024
EPS = 1e-23

VPAD = 1024
BBLK = 1024
NCTX = BATCH * NGRAM

ROWW = 128
DPS = 4
NSTEP = DIM // DPS


def _softplus(x):
    return jnp.log(1.0 + jnp.exp(x))


def _sp_eps_from_w(w):
    sp_a = w * (1.0 - 0.5 * w)
    sp_b = jnp.log(1.0 + w)
    return jnp.where(w <= 6.7379470e-3, sp_a, sp_b) + EPS


def _make_sc_gather():
    info = plsc.get_sparse_core_info()
    nw = info.num_cores * info.num_subcores
    ctx_per_w = NCTX // nw
    voc_per_w = VPAD // nw

    mesh = plsc.VectorSubcoreMesh(core_axis_name="c", subcore_axis_name="s")

    @functools.partial(
        pl.kernel,
        out_type=[
            jax.ShapeDtypeStruct((NCTX, ROWW), jnp.float32),
            jax.ShapeDtypeStruct((VPAD, ROWW), jnp.float32),
        ],
        mesh=mesh,
        scratch_types=[
            pltpu.VMEM((ctx_per_w,), jnp.int32),
            pltpu.VMEM((voc_per_w,), jnp.int32),
            pltpu.VMEM((ctx_per_w, ROWW), jnp.float32),
            pltpu.VMEM((voc_per_w, ROWW), jnp.float32),
            pltpu.SemaphoreType.DMA,
        ],
    )
    def gather_k(table_hbm, xidx_hbm, vidx_hbm, ctx_hbm, wbv_hbm,
                 xi_v, vi_v, ctx_v, wbv_v, sem):
        wid = lax.axis_index("s") * info.num_cores + lax.axis_index("c")
        cbase = wid * ctx_per_w
        vbase = wid * voc_per_w
        pltpu.sync_copy(xidx_hbm.at[pl.ds(cbase, ctx_per_w)], xi_v)
        pltpu.sync_copy(vidx_hbm.at[pl.ds(vbase, voc_per_w)], vi_v)
        c1 = pltpu.async_copy(table_hbm.at[xi_v], ctx_v, sem)
        c2 = pltpu.async_copy(table_hbm.at[vi_v], wbv_v, sem)
        c1.wait()
        c2.wait()
        pltpu.sync_copy(ctx_v, ctx_hbm.at[pl.ds(cbase, ctx_per_w)])
        pltpu.sync_copy(wbv_v, wbv_hbm.at[pl.ds(vbase, voc_per_w)])

    return gather_k


@functools.cache
def _sc_gather_fn():
    return _make_sc_gather()


def _score_body(ctx_ref, wbv_ref, pmw_ref, pmb_ref, pdw_ref, pdb_ref,
                bias_ref, out_ref, E1t_s, Ei1t_s, E2_s, Ei2_s):
    i = pl.program_id(0)

    @pl.when(i == 0)
    def _prep():
        wbt = wbv_ref[:, 0:2 * DIM].T
        z1 = wbt[0:DIM, :]
        Z1 = z1 + _softplus(wbt[DIM:2 * DIM, :])
        E1t_s[...] = jnp.exp(Z1).reshape(DIM, 1, VPAD)
        Ei1t_s[...] = jnp.exp(-z1).reshape(DIM, 1, VPAD)
        zacc = None
        dacc = None
        for g in range(NGRAM):
            cm = ctx_ref[:, g, 0:DIM]
            cd = ctx_ref[:, g, DIM:2 * DIM]
            mn = cm * pmw_ref[g:g + 1, :] + pmb_ref[g:g + 1, :]
            dl = _softplus(cd * pdw_ref[g:g + 1, :] + pdb_ref[g:g + 1, :])
            zacc = mn if zacc is None else zacc + mn
            dacc = dl if dacc is None else dacc + dl
        z2 = zacc * (1.0 / NGRAM)
        d2 = dacc * (1.0 / NGRAM)
        E2_s[...] = jnp.exp(z2 + _softplus(d2))
        Ei2_s[...] = jnp.exp(-z2)

    E2blk = E2_s[...]
    Ei2blk = Ei2_s[...]
    lane = lax.broadcasted_iota(jnp.int32, (BBLK, DIM), 1)

    sps = []
    for k in range(DPS):
        d = i * DPS + k
        E1r = E1t_s[d]
        Ei1r = Ei1t_s[d]
        sel = lane == d
        E2c = jnp.sum(jnp.where(sel, E2blk, 0.0), axis=1, keepdims=True)
        Ei2c = jnp.sum(jnp.where(sel, Ei2blk, 0.0), axis=1, keepdims=True)
        w = jnp.minimum(E1r, E2c) * jnp.minimum(Ei1r, Ei2c)
        sps.append(_sp_eps_from_w(w))

    p01 = jnp.maximum(sps[0] * sps[1], 1.2e-38) * 4294967296.0
    p23 = jnp.maximum(sps[2] * sps[3], 1.2e-38) * 4294967296.0
    acc = jnp.log(jnp.maximum(p01 * p23, 1.2e-38))

    @pl.when(i == 0)
    def _():
        out_ref[...] = acc

    @pl.when(i != 0)
    def _():
        out_ref[...] += acc

    @pl.when(i == NSTEP - 1)
    def _():
        dec = out_ref[...] + (bias_ref[...] - NSTEP * 64.0 * 0.6931471805599453)
        vlane = lax.broadcasted_iota(jnp.int32, (BBLK, VPAD), 1)
        dec = jnp.where(vlane < VOCAB, dec, -1e30)
        m = jnp.max(dec, axis=1, keepdims=True)
        lse = jnp.log(jnp.sum(jnp.exp(dec - m), axis=1, keepdims=True)) + m
        out_ref[...] = dec - lse


def kernel(x, word_boxes, embedding_bias, pos_delta_w, pos_delta_b,
           pos_min_w, pos_min_b):
    table = word_boxes.reshape(VOCAB * NGRAM, 2 * DIM).astype(jnp.float32)
    table = jnp.pad(table, ((0, 0), (0, ROWW - 2 * DIM)))

    xflat = x.reshape(-1).astype(jnp.int32)
    vidx = jnp.pad(jnp.arange(VOCAB, dtype=jnp.int32) * NGRAM,
                   (0, VPAD - VOCAB))

    ctx_rows, wbv_rows = _sc_gather_fn()(table, xflat, vidx)
    ctx4 = ctx_rows.reshape(BATCH, NGRAM, ROWW)

    bias_row = jnp.pad(embedding_bias.reshape(1, VOCAB),
                       ((0, 0), (0, VPAD - VOCAB)))

    out = pl.pallas_call(
        _score_body,
        grid=(NSTEP,),
        in_specs=[
            pl.BlockSpec((BATCH, NGRAM, ROWW), lambda d: (0, 0, 0)),
            pl.BlockSpec((VPAD, ROWW), lambda d: (0, 0)),
            pl.BlockSpec((NGRAM, DIM), lambda d: (0, 0)),
            pl.BlockSpec((NGRAM, DIM), lambda d: (0, 0)),
            pl.BlockSpec((NGRAM, DIM), lambda d: (0, 0)),
            pl.BlockSpec((NGRAM, DIM), lambda d: (0, 0)),
            pl.BlockSpec((1, VPAD), lambda d: (0, 0)),
        ],
        out_specs=pl.BlockSpec((BBLK, VPAD), lambda d: (0, 0)),
        out_shape=jax.ShapeDtypeStruct((BATCH, VPAD), jnp.float32),
        scratch_shapes=[
            pltpu.VMEM((DIM, 1, VPAD), jnp.float32),
            pltpu.VMEM((DIM, 1, VPAD), jnp.float32),
            pltpu.VMEM((BATCH, DIM), jnp.float32),
            pltpu.VMEM((BATCH, DIM), jnp.float32),
        ],
        compiler_params=pltpu.CompilerParams(
            dimension_semantics=("arbitrary",)),
    )(ctx4, wbv_rows, pos_min_w, pos_min_b, pos_delta_w, pos_delta_b,
      bias_row)

    return out[:, :VOCAB]


# speedup vs baseline: 1.2076x; 1.2076x over previous
import functools

import jax
import jax.numpy as jnp
from jax import lax
from jax.experimental import pallas as pl
from jax.experimental.pallas import tpu as pltpu
from jax.experimental.pallas import tpu_sc as plsc

VOCAB = 1000
NGRAM = 4
DIM = 32
BATCH = 1024
EPS = 1e-23

VPAD = 1024
BBLK = 1024
NCTX = BATCH * NGRAM

ROWW = 128
DPS = 8
NSTEP = DIM // DPS


def _softplus(x):
    return jnp.log(1.0 + jnp.exp(x))


def _sp_from_w(w):
    return jnp.where(w <= 6.7379470e-3, w, jnp.log(1.0 + w))


def _make_sc_gather():
    info = plsc.get_sparse_core_info()
    nw = info.num_cores * info.num_subcores
    ctx_per_w = NCTX // nw
    voc_per_w = VPAD // nw

    mesh = plsc.VectorSubcoreMesh(core_axis_name="c", subcore_axis_name="s")

    @functools.partial(
        pl.kernel,
        out_type=[
            jax.ShapeDtypeStruct((NCTX, ROWW), jnp.float32),
            jax.ShapeDtypeStruct((VPAD, ROWW), jnp.float32),
        ],
        mesh=mesh,
        scratch_types=[
            pltpu.VMEM((ctx_per_w,), jnp.int32),
            pltpu.VMEM((voc_per_w,), jnp.int32),
            pltpu.VMEM((ctx_per_w, ROWW), jnp.float32),
            pltpu.VMEM((voc_per_w, ROWW), jnp.float32),
            pltpu.SemaphoreType.DMA,
        ],
    )
    def gather_k(table_hbm, xidx_hbm, vidx_hbm, ctx_hbm, wbv_hbm,
                 xi_v, vi_v, ctx_v, wbv_v, sem):
        wid = lax.axis_index("s") * info.num_cores + lax.axis_index("c")
        cbase = wid * ctx_per_w
        vbase = wid * voc_per_w
        pltpu.sync_copy(xidx_hbm.at[pl.ds(cbase, ctx_per_w)], xi_v)
        pltpu.sync_copy(vidx_hbm.at[pl.ds(vbase, voc_per_w)], vi_v)
        c1 = pltpu.async_copy(table_hbm.at[xi_v], ctx_v, sem)
        c2 = pltpu.async_copy(table_hbm.at[vi_v], wbv_v, sem)
        c1.wait()
        c2.wait()
        pltpu.sync_copy(ctx_v, ctx_hbm.at[pl.ds(cbase, ctx_per_w)])
        pltpu.sync_copy(wbv_v, wbv_hbm.at[pl.ds(vbase, voc_per_w)])

    return gather_k


@functools.cache
def _sc_gather_fn():
    return _make_sc_gather()


def _score_body(ctx_ref, wbv_ref, pmw_ref, pmb_ref, pdw_ref, pdb_ref,
                bias_ref, out_ref, E1t_s, Ei1t_s, E2_s, Ei2_s):
    i = pl.program_id(0)

    @pl.when(i == 0)
    def _prep():
        wbt = wbv_ref[:, 0:2 * DIM].T
        z1 = wbt[0:DIM, :]
        Z1 = z1 + _softplus(wbt[DIM:2 * DIM, :])
        E1t_s[...] = jnp.exp(Z1).reshape(DIM, 1, VPAD)
        Ei1t_s[...] = jnp.exp(-z1).reshape(DIM, 1, VPAD)
        zacc = None
        dacc = None
        for g in range(NGRAM):
            cm = ctx_ref[:, g, 0:DIM]
            cd = ctx_ref[:, g, DIM:2 * DIM]
            mn = cm * pmw_ref[g:g + 1, :] + pmb_ref[g:g + 1, :]
            dl = _softplus(cd * pdw_ref[g:g + 1, :] + pdb_ref[g:g + 1, :])
            zacc = mn if zacc is None else zacc + mn
            dacc = dl if dacc is None else dacc + dl
        z2 = zacc * (1.0 / NGRAM)
        d2 = dacc * (1.0 / NGRAM)
        E2_s[...] = jnp.exp(z2 + _softplus(d2))
        Ei2_s[...] = jnp.exp(-z2)

    E2blk = E2_s[...]
    Ei2blk = Ei2_s[...]
    lane = lax.broadcasted_iota(jnp.int32, (BBLK, DIM), 1)

    sps = []
    for k in range(DPS):
        d = i * DPS + k
        E1r = E1t_s[d]
        Ei1r = Ei1t_s[d]
        sel = lane == d
        E2c = jnp.sum(jnp.where(sel, E2blk, 0.0), axis=1, keepdims=True)
        Ei2c = jnp.sum(jnp.where(sel, Ei2blk, 0.0), axis=1, keepdims=True)
        w = jnp.minimum(E1r, E2c) * jnp.minimum(Ei1r, Ei2c)
        sps.append(_sp_from_w(w))

    acc = None
    for k in range(0, DPS, 2):
        pair = jnp.log(jnp.maximum(sps[k] * sps[k + 1], 1.2e-38))
        acc = pair if acc is None else acc + pair

    @pl.when(i == 0)
    def _():
        out_ref[...] = acc

    @pl.when(i != 0)
    def _():
        out_ref[...] += acc

    @pl.when(i == NSTEP - 1)
    def _():
        dec = out_ref[...] + bias_ref[...]
        vlane = lax.broadcasted_iota(jnp.int32, (BBLK, VPAD), 1)
        dec = jnp.where(vlane < VOCAB, dec, -1e30)
        m = jnp.max(dec, axis=1, keepdims=True)
        lse = jnp.log(jnp.sum(jnp.exp(dec - m), axis=1, keepdims=True)) + m
        out_ref[...] = dec - lse


def kernel(x, word_boxes, embedding_bias, pos_delta_w, pos_delta_b,
           pos_min_w, pos_min_b):
    table = word_boxes.reshape(VOCAB * NGRAM, 2 * DIM).astype(jnp.float32)
    table = jnp.pad(table, ((0, 0), (0, ROWW - 2 * DIM)))

    xflat = x.reshape(-1).astype(jnp.int32)
    vidx = jnp.pad(jnp.arange(VOCAB, dtype=jnp.int32) * NGRAM,
                   (0, VPAD - VOCAB))

    ctx_rows, wbv_rows = _sc_gather_fn()(table, xflat, vidx)
    ctx4 = ctx_rows.reshape(BATCH, NGRAM, ROWW)

    bias_row = jnp.pad(embedding_bias.reshape(1, VOCAB),
                       ((0, 0), (0, VPAD - VOCAB)))

    out = pl.pallas_call(
        _score_body,
        grid=(NSTEP,),
        in_specs=[
            pl.BlockSpec((BATCH, NGRAM, ROWW), lambda d: (0, 0, 0)),
            pl.BlockSpec((VPAD, ROWW), lambda d: (0, 0)),
            pl.BlockSpec((NGRAM, DIM), lambda d: (0, 0)),
            pl.BlockSpec((NGRAM, DIM), lambda d: (0, 0)),
            pl.BlockSpec((NGRAM, DIM), lambda d: (0, 0)),
            pl.BlockSpec((NGRAM, DIM), lambda d: (0, 0)),
            pl.BlockSpec((1, VPAD), lambda d: (0, 0)),
        ],
        out_specs=pl.BlockSpec((BBLK, VPAD), lambda d: (0, 0)),
        out_shape=jax.ShapeDtypeStruct((BATCH, VPAD), jnp.float32),
        scratch_shapes=[
            pltpu.VMEM((DIM, 1, VPAD), jnp.float32),
            pltpu.VMEM((DIM, 1, VPAD), jnp.float32),
            pltpu.VMEM((BATCH, DIM), jnp.float32),
            pltpu.VMEM((BATCH, DIM), jnp.float32),
        ],
        compiler_params=pltpu.CompilerParams(
            dimension_semantics=("arbitrary",)),
    )(ctx4, wbv_rows, pos_min_w, pos_min_b, pos_delta_w, pos_delta_b,
      bias_row)

    return out[:, :VOCAB]
